# baseline (speedup 1.0000x reference)
import functools

import jax
import jax.numpy as jnp
from jax.experimental import pallas as pl
from jax.experimental.pallas import tpu as pltpu

_TIMESTEPS = 100


def _tables():
    scale = 1000.0 / _TIMESTEPS
    betas = jnp.linspace(scale * 0.0001, scale * 0.02, _TIMESTEPS)
    alphas_cumprod = jnp.cumprod(1.0 - betas)
    return (jnp.sqrt(alphas_cumprod).astype(jnp.float32),
            jnp.sqrt(1.0 - alphas_cumprod).astype(jnp.float32))


def _body(t_ref, ta_ref, tb_ref, x_ref, n_ref, o_ref):
    b = pl.program_id(0)
    tt = t_ref[b]
    ca = ta_ref[tt]
    cb = tb_ref[tt]
    o_ref[...] = ca * x_ref[...] + cb * n_ref[...]


@jax.jit
def kernel(x_start, t, noise):
    B, C, H, W = x_start.shape
    ta, tb = _tables()
    x2 = x_start.reshape(B, 1, C * H * W)
    n2 = noise.reshape(B, 1, C * H * W)
    n_chunks = 6
    chunk = (C * H * W) // n_chunks

    grid_spec = pltpu.PrefetchScalarGridSpec(
        num_scalar_prefetch=3,
        grid=(B, n_chunks),
        in_specs=[
            pl.BlockSpec((1, 1, chunk), lambda b, j, *_: (b, 0, j)),
            pl.BlockSpec((1, 1, chunk), lambda b, j, *_: (b, 0, j)),
        ],
        out_specs=pl.BlockSpec((1, 1, chunk), lambda b, j, *_: (b, 0, j)),
    )
    out = pl.pallas_call(
        _body,
        grid_spec=grid_spec,
        out_shape=jax.ShapeDtypeStruct((B, 1, C * H * W), jnp.float32),
    )(t, ta, tb, x2, n2)
    return out.reshape(B, C, H, W)


# speedup vs baseline: 8.4639x; 8.4639x over previous
import functools

import jax
import jax.numpy as jnp
from jax.experimental import pallas as pl
from jax.experimental.pallas import tpu as pltpu

_TIMESTEPS = 100


def _tables():
    scale = 1000.0 / _TIMESTEPS
    betas = jnp.linspace(scale * 0.0001, scale * 0.02, _TIMESTEPS)
    alphas_cumprod = jnp.cumprod(1.0 - betas)
    return (jnp.sqrt(alphas_cumprod).astype(jnp.float32),
            jnp.sqrt(1.0 - alphas_cumprod).astype(jnp.float32))


def _body(t_ref, ta_ref, tb_ref, x_ref, n_ref, o_ref):
    b = pl.program_id(0)
    tt = t_ref[b]
    ca = ta_ref[tt]
    cb = tb_ref[tt]
    o_ref[...] = ca * x_ref[...] + cb * n_ref[...]


@jax.jit
def kernel(x_start, t, noise):
    B, C, H, W = x_start.shape
    ta, tb = _tables()

    grid_spec = pltpu.PrefetchScalarGridSpec(
        num_scalar_prefetch=3,
        grid=(B, C),
        in_specs=[
            pl.BlockSpec((1, 1, H, W), lambda b, c, *_: (b, c, 0, 0)),
            pl.BlockSpec((1, 1, H, W), lambda b, c, *_: (b, c, 0, 0)),
        ],
        out_specs=pl.BlockSpec((1, 1, H, W), lambda b, c, *_: (b, c, 0, 0)),
    )
    return pl.pallas_call(
        _body,
        grid_spec=grid_spec,
        out_shape=jax.ShapeDtypeStruct((B, C, H, W), jnp.float32),
    )(t, ta, tb, x_start, noise)


# speedup vs baseline: 10.3656x; 1.2247x over previous
import functools

import jax
import jax.numpy as jnp
from jax.experimental import pallas as pl
from jax.experimental.pallas import tpu as pltpu

_TIMESTEPS = 100


def _tables():
    scale = 1000.0 / _TIMESTEPS
    betas = jnp.linspace(scale * 0.0001, scale * 0.02, _TIMESTEPS)
    alphas_cumprod = jnp.cumprod(1.0 - betas)
    return (jnp.sqrt(alphas_cumprod).astype(jnp.float32),
            jnp.sqrt(1.0 - alphas_cumprod).astype(jnp.float32))


def _body(t_ref, ta_ref, tb_ref, x_ref, n_ref, o_ref):
    b = pl.program_id(0)
    tt = t_ref[b]
    ca = ta_ref[tt]
    cb = tb_ref[tt]
    o_ref[...] = ca * x_ref[...] + cb * n_ref[...]


@jax.jit
def kernel(x_start, t, noise):
    B, C, H, W = x_start.shape
    ta, tb = _tables()

    grid_spec = pltpu.PrefetchScalarGridSpec(
        num_scalar_prefetch=3,
        grid=(B,),
        in_specs=[
            pl.BlockSpec((1, C, H, W), lambda b, *_: (b, 0, 0, 0)),
            pl.BlockSpec((1, C, H, W), lambda b, *_: (b, 0, 0, 0)),
        ],
        out_specs=pl.BlockSpec((1, C, H, W), lambda b, *_: (b, 0, 0, 0)),
    )
    return pl.pallas_call(
        _body,
        grid_spec=grid_spec,
        out_shape=jax.ShapeDtypeStruct((B, C, H, W), jnp.float32),
    )(t, ta, tb, x_start, noise)
